# baseline (speedup 1.0000x reference)
import functools

import jax
import jax.numpy as jnp
from jax import lax
from jax.experimental import pallas as pl
from jax.experimental.pallas import tpu as pltpu
from jax.experimental.pallas import tpu_sc as plsc

_DIM = 64
_SCALE = float(_DIM) ** 0.5
_L = 16
_NC, _NS = 2, 16
_NW = _NC * _NS


@functools.partial(jax.jit, static_argnames=("chunk",))
def _embed_gather(idx_flat, table, chunk=1600):
    n = idx_flat.shape[0]
    bpw = n // _NW
    nchunk = bpw // chunk
    mesh = plsc.VectorSubcoreMesh(core_axis_name="c", subcore_axis_name="s")

    @functools.partial(
        pl.kernel,
        out_type=jax.ShapeDtypeStruct((n, _DIM), jnp.float32),
        mesh=mesh,
        scratch_types=[
            pltpu.VMEM((chunk,), jnp.int32),
            pltpu.VMEM((chunk, _DIM), jnp.float32),
            pltpu.SemaphoreType.DMA,
        ],
        compiler_params=pltpu.CompilerParams(use_tc_tiling_on_sc=False),
    )
    def k(idx_hbm, tab_hbm, out_hbm, idx_v, rows_v, sem):
        wid = lax.axis_index("s") * _NC + lax.axis_index("c")
        base = wid * bpw

        def do_chunk(c, carry):
            off = base + c * chunk
            pltpu.sync_copy(idx_hbm.at[pl.ds(off, chunk)], idx_v)
            pltpu.async_copy(tab_hbm.at[idx_v], rows_v, sem).wait()

            def scale_row(r, carry2):
                for j in range(_DIM // _L):
                    sl = (r, pl.ds(j * _L, _L))
                    rows_v[sl] = rows_v[sl] * _SCALE
                return carry2

            lax.fori_loop(0, chunk, scale_row, 0, unroll=2)
            pltpu.sync_copy(rows_v, out_hbm.at[pl.ds(off, chunk)])
            return carry

        lax.fori_loop(0, nchunk, do_chunk, 0)

    return k(idx_flat, table)


def kernel(input_ids, table):
    n_tok = input_ids.shape[0] * input_ids.shape[1]
    idx_flat = input_ids.reshape(n_tok)
    out = _embed_gather(idx_flat, table)
    return out.reshape(input_ids.shape + (_DIM,))
